# Initial kernel scaffold; baseline (speedup 1.0000x reference)
import jax
import jax.numpy as jnp
from jax.experimental import pallas as pl


def kernel(x, query_pos, grid_to_query_edges, W1, b1, W2, b2, W3, b3, P1, pb1, P2, pb2):
    raise NotImplementedError("write your pallas kernel here")


# speedup vs baseline: 2.9826x; 2.9826x over previous
import functools

import jax
import jax.numpy as jnp
from jax import lax
from jax.experimental import pallas as pl
from jax.experimental.pallas import tpu as pltpu
import jax.experimental.pallas.tpu_sc as plsc

INPUT_DIM = 128
HIDDEN = 32
POS_DIM = 4 * HIDDEN
N_QUERY = 10000
N_EDGES = 160000
SEQLEN = 32768

NC, NS = 2, 16
NW = NC * NS
CHUNK = 128
CHUNKS_PER_W = 5120 // CHUNK
E_PAD = NW * CHUNKS_PER_W * CHUNK
ROWS_PER_W = E_PAD // NW
H3W = 128
NROW = 10240
ROWS_PER_TILE = NROW // NS

MLP_BLK = 1024
FIN_BLK = 1000


def _gelu(x):
    return 0.5 * x * (1.0 + lax.erf(x * 0.7071067811865476))


def _pos_body(qp_ref, o_ref):
    k32 = lax.broadcasted_iota(jnp.int32, o_ref.shape, 1) % 32
    i = (k32 % 16).astype(jnp.float32)
    omega = jnp.exp(i * (-2.0 / 32.0 * jnp.log(10000.0)))
    ph = qp_ref[...] * omega
    o_ref[...] = jnp.where(k32 < 16, jnp.sin(ph), jnp.cos(ph))


def _pos_embed(qp_exp):
    return pl.pallas_call(
        _pos_body,
        out_shape=jax.ShapeDtypeStruct((N_QUERY, POS_DIM), jnp.float32),
    )(qp_exp)


def _gather_body(xf_hbm, pos_hbm, gidx_hbm, qidx_hbm, outa_hbm, outb_hbm,
                 gi_v, qi_v, ra_v, rb_v, sem_a, sem_b):
    wid = lax.axis_index("s") * NC + lax.axis_index("c")

    def step(c, _):
        base = wid * ROWS_PER_W + c * CHUNK
        pltpu.sync_copy(gidx_hbm.at[pl.ds(base, CHUNK)], gi_v)
        pltpu.sync_copy(qidx_hbm.at[pl.ds(base, CHUNK)], qi_v)
        cpa = pltpu.async_copy(xf_hbm.at[gi_v], ra_v, sem_a)
        cpb = pltpu.async_copy(pos_hbm.at[qi_v], rb_v, sem_b)
        cpa.wait()
        cpb.wait()
        pltpu.sync_copy(ra_v, outa_hbm.at[pl.ds(base, CHUNK)])
        pltpu.sync_copy(rb_v, outb_hbm.at[pl.ds(base, CHUNK)])
        return 0

    lax.fori_loop(0, CHUNKS_PER_W, step, 0)


@functools.cache
def _gather():
    return pl.kernel(
        _gather_body,
        out_type=(jax.ShapeDtypeStruct((E_PAD, INPUT_DIM), jnp.float32),
                  jax.ShapeDtypeStruct((E_PAD, POS_DIM), jnp.float32)),
        mesh=plsc.VectorSubcoreMesh(core_axis_name="c", subcore_axis_name="s",
                                    num_cores=NC, num_subcores=NS),
        scratch_types=[
            pltpu.VMEM((CHUNK,), jnp.int32),
            pltpu.VMEM((CHUNK,), jnp.int32),
            pltpu.VMEM((CHUNK, INPUT_DIM), jnp.float32),
            pltpu.VMEM((CHUNK, POS_DIM), jnp.float32),
            pltpu.SemaphoreType.DMA,
            pltpu.SemaphoreType.DMA,
        ],
    )


def _mlp_body(a_ref, b_ref, w1a_ref, w1b_ref, b1_ref, w2_ref, b2_ref,
              w3_ref, b3_ref, o_ref):
    h = jnp.dot(a_ref[...], w1a_ref[...], preferred_element_type=jnp.float32)
    h = h + jnp.dot(b_ref[...], w1b_ref[...], preferred_element_type=jnp.float32)
    h = _gelu(h + b1_ref[...])
    h = _gelu(jnp.dot(h, w2_ref[...], preferred_element_type=jnp.float32)
              + b2_ref[...])
    h = jnp.dot(h, w3_ref[...], preferred_element_type=jnp.float32) + b3_ref[...]
    o_ref[...] = h


def _edge_mlp(h0a, h0b, W1a, W1b, b1, W2, b2, W3, b3):
    nblk = E_PAD // MLP_BLK
    full = lambda shape: pl.BlockSpec(shape, lambda i: (0, 0))
    return pl.pallas_call(
        _mlp_body,
        grid=(nblk,),
        in_specs=[
            pl.BlockSpec((MLP_BLK, INPUT_DIM), lambda i: (i, 0)),
            pl.BlockSpec((MLP_BLK, POS_DIM), lambda i: (i, 0)),
            full((INPUT_DIM, 512)), full((POS_DIM, 512)), full((1, 512)),
            full((512, 256)), full((1, 256)),
            full((256, 128)), full((1, 128)),
        ],
        out_specs=pl.BlockSpec((MLP_BLK, H3W), lambda i: (i, 0)),
        out_shape=jax.ShapeDtypeStruct((E_PAD, H3W), jnp.float32),
    )(h0a, h0b, W1a, W1b, b1, W2, b2, W3, b3)


CROWS = NROW // 128
CROWS_PER_TILE = 8


def _scatter_body(h3_hbm, qidx_hbm, zeros_hbm, sums_hbm, cnt_hbm,
                  qi_v, rows_v, cnt2_v, ridx_v, acc_sh, cntacc_sh):
    cid = lax.axis_index("c")
    sid = lax.axis_index("s")
    wid = sid * NC + cid
    my = pl.ds(sid * ROWS_PER_TILE, ROWS_PER_TILE)
    myc = pl.ds(sid * CROWS_PER_TILE, CROWS_PER_TILE)
    pltpu.sync_copy(zeros_hbm, acc_sh.at[my])
    pltpu.sync_copy(zeros_hbm.at[pl.ds(0, CROWS)], cnt2_v)

    @pl.when(sid < CROWS // CROWS_PER_TILE)
    def _():
        pltpu.sync_copy(zeros_hbm.at[pl.ds(0, CROWS_PER_TILE)],
                        cntacc_sh.at[myc])
    for j in range(CROWS // 16):
        ridx_v[pl.ds(j * 16, 16)] = lax.iota(jnp.int32, 16) + (j * 16)
    plsc.subcore_barrier()
    ones16 = jnp.ones((16,), jnp.float32)

    def step(c, _):
        base = wid * ROWS_PER_W + c * CHUNK
        pltpu.sync_copy(qidx_hbm.at[pl.ds(base, CHUNK)], qi_v)
        pltpu.sync_copy(h3_hbm.at[pl.ds(base, CHUNK)], rows_v)
        pltpu.sync_copy(rows_v, acc_sh.at[qi_v], add=True)
        for j in range(CHUNK // 16):
            idx16 = qi_v[pl.ds(j * 16, 16)]
            plsc.addupdate_scatter(
                cnt2_v, [lax.shift_right_logical(idx16, 7),
                         lax.bitwise_and(idx16, 127)], ones16)
        return 0

    lax.fori_loop(0, CHUNKS_PER_W, step, 0)
    pltpu.sync_copy(cnt2_v, cntacc_sh.at[ridx_v], add=True)
    plsc.subcore_barrier()
    pltpu.sync_copy(acc_sh.at[my], sums_hbm.at[cid].at[my])

    @pl.when(sid < CROWS // CROWS_PER_TILE)
    def _():
        pltpu.sync_copy(cntacc_sh.at[myc], cnt_hbm.at[cid].at[myc])


@functools.cache
def _scatter():
    return pl.kernel(
        _scatter_body,
        out_type=(jax.ShapeDtypeStruct((NC, NROW, H3W), jnp.float32),
                  jax.ShapeDtypeStruct((NC, CROWS, 128), jnp.float32)),
        mesh=plsc.VectorSubcoreMesh(core_axis_name="c", subcore_axis_name="s",
                                    num_cores=NC, num_subcores=NS),
        scratch_types=[
            pltpu.VMEM((CHUNK,), jnp.int32),
            pltpu.VMEM((CHUNK, H3W), jnp.float32),
            pltpu.VMEM((CROWS, 128), jnp.float32),
            pltpu.VMEM((CROWS,), jnp.int32),
            pltpu.VMEM_SHARED((NROW, H3W), jnp.float32),
            pltpu.VMEM_SHARED((CROWS, 128), jnp.float32),
        ],
        compiler_params=pltpu.CompilerParams(needs_layout_passes=False),
    )


def _fin_body(p0_ref, p1_ref, c0_ref, c1_ref, P1_ref, pb1_ref, P2_ref,
              pb2_ref, o_ref):
    sums = p0_ref[...] + p1_ref[...]
    cnt = c0_ref[...] + c1_ref[...]
    mean = jnp.where(cnt > 0, sums / jnp.maximum(cnt, 1.0), 0.0)
    h = _gelu(jnp.dot(mean, P1_ref[...], preferred_element_type=jnp.float32)
              + pb1_ref[...])
    o_ref[...] = (jnp.dot(h, P2_ref[...], preferred_element_type=jnp.float32)
                  + pb2_ref[...])


def _finalize(p0, p1, c0, c1, P1, pb1, P2, pb2):
    nblk = N_QUERY // FIN_BLK
    full = lambda shape: pl.BlockSpec(shape, lambda i: (0, 0))
    return pl.pallas_call(
        _fin_body,
        grid=(nblk,),
        in_specs=[
            pl.BlockSpec((FIN_BLK, H3W), lambda i: (i, 0)),
            pl.BlockSpec((FIN_BLK, H3W), lambda i: (i, 0)),
            pl.BlockSpec((FIN_BLK, 1), lambda i: (i, 0)),
            pl.BlockSpec((FIN_BLK, 1), lambda i: (i, 0)),
            full((128, 256)), full((1, 256)), full((256, 4)), full((1, 4)),
        ],
        out_specs=pl.BlockSpec((FIN_BLK, 4), lambda i: (i, 0)),
        out_shape=jax.ShapeDtypeStruct((N_QUERY, 4), jnp.float32),
    )(p0, p1, c0, c1, P1, pb1, P2, pb2)


def kernel(x, query_pos, grid_to_query_edges, W1, b1, W2, b2, W3, b3,
           P1, pb1, P2, pb2):
    xf = x.reshape(-1, x.shape[-1])
    qp = query_pos / 100.0 - 1.0
    qp4 = jnp.concatenate([qp, jnp.ones((N_QUERY, 1), qp.dtype)], axis=1)
    qp_exp = jnp.repeat(qp4, 32, axis=1)

    pos = _pos_embed(qp_exp)

    pad = E_PAD - N_EDGES
    qidx = grid_to_query_edges[:, 0]
    gidx_p = jnp.concatenate([grid_to_query_edges[:, 1],
                              jnp.zeros((pad,), jnp.int32)])
    qidx_g = jnp.concatenate([qidx, jnp.zeros((pad,), jnp.int32)])
    qidx_s = jnp.concatenate([qidx, jnp.full((pad,), N_QUERY, jnp.int32)])

    h0a, h0b = _gather()(xf, pos, gidx_p, qidx_g)

    h3 = _edge_mlp(h0a, h0b, W1[:INPUT_DIM], W1[INPUT_DIM:],
                   b1.reshape(1, -1), W2, b2.reshape(1, -1),
                   W3, b3.reshape(1, -1))

    zeros = jnp.zeros((ROWS_PER_TILE, H3W), jnp.float32)
    sums_p, cnt_p = _scatter()(h3, qidx_s, zeros)

    return _finalize(sums_p[0], sums_p[1],
                     cnt_p[0].reshape(NROW, 1), cnt_p[1].reshape(NROW, 1),
                     P1, pb1.reshape(1, -1), P2, pb2.reshape(1, -1))


# speedup vs baseline: 3.5725x; 1.1978x over previous
import functools

import jax
import jax.numpy as jnp
from jax import lax
from jax.experimental import pallas as pl
from jax.experimental.pallas import tpu as pltpu
import jax.experimental.pallas.tpu_sc as plsc

INPUT_DIM = 128
HIDDEN = 32
POS_DIM = 4 * HIDDEN
N_QUERY = 10000
N_EDGES = 160000
SEQLEN = 32768

NC, NS = 2, 16
NW = NC * NS
CHUNK = 128
CHUNKS_PER_W = 5120 // CHUNK
E_PAD = NW * CHUNKS_PER_W * CHUNK
ROWS_PER_W = E_PAD // NW
H3W = 128
NROW = 10240
ROWS_PER_TILE = NROW // NS

MLP_BLK = 1024
FIN_BLK = 1000


def _gelu(x):
    return 0.5 * x * (1.0 + lax.erf(x * 0.7071067811865476))


def _pos_body(qp_ref, o_ref):
    k32 = lax.broadcasted_iota(jnp.int32, o_ref.shape, 1) % 32
    i = (k32 % 16).astype(jnp.float32)
    omega = jnp.exp(i * (-2.0 / 32.0 * jnp.log(10000.0)))
    ph = qp_ref[...] * omega
    o_ref[...] = jnp.where(k32 < 16, jnp.sin(ph), jnp.cos(ph))


def _pos_embed(qp_exp):
    return pl.pallas_call(
        _pos_body,
        out_shape=jax.ShapeDtypeStruct((N_QUERY, POS_DIM), jnp.float32),
    )(qp_exp)


NBUF = 2


def _gather_body(xf_hbm, pos_hbm, gidx_hbm, qidx_hbm, outa_hbm, outb_hbm,
                 gi_v, qi_v, ra_v, rb_v, gsem0, gsem1, wsem0, wsem1):
    gsem = [gsem0, gsem1]
    wsem = [wsem0, wsem1]
    wid = lax.axis_index("s") * NC + lax.axis_index("c")
    wbase = wid * ROWS_PER_W
    pltpu.sync_copy(gidx_hbm.at[pl.ds(wbase, ROWS_PER_W)], gi_v)
    pltpu.sync_copy(qidx_hbm.at[pl.ds(wbase, ROWS_PER_W)], qi_v)

    def g_copies(c, b):
        sl = pl.ds(c * CHUNK, CHUNK)
        return (pltpu.make_async_copy(xf_hbm.at[gi_v.at[sl]], ra_v.at[b],
                                      gsem[b]),
                pltpu.make_async_copy(pos_hbm.at[qi_v.at[sl]], rb_v.at[b],
                                      gsem[b]))

    def w_copies(c, b):
        sl = pl.ds(wbase + c * CHUNK, CHUNK)
        return (pltpu.make_async_copy(ra_v.at[b], outa_hbm.at[sl], wsem[b]),
                pltpu.make_async_copy(rb_v.at[b], outb_hbm.at[sl], wsem[b]))

    def start(c, b):
        for cp in g_copies(c, b):
            cp.start()

    def drain_gather(c, b):
        for cp in g_copies(c, b):
            cp.wait()

    def writeback(c, b):
        for cp in w_copies(c, b):
            cp.start()

    def drain_write(c, b):
        for cp in w_copies(c, b):
            cp.wait()

    for b in range(NBUF):
        start(b, b)

    def step(i, _):
        for b in range(NBUF):
            c = i * NBUF + b
            drain_gather(c, b)
            writeback(c, b)

            @pl.when(c + NBUF < CHUNKS_PER_W)
            def _():
                drain_write(c, b)
                start(c + NBUF, b)

        return 0

    lax.fori_loop(0, CHUNKS_PER_W // NBUF, step, 0)
    for b in range(NBUF):
        drain_write(CHUNKS_PER_W - NBUF + b, b)


@functools.cache
def _gather():
    return pl.kernel(
        _gather_body,
        out_type=(jax.ShapeDtypeStruct((E_PAD, INPUT_DIM), jnp.float32),
                  jax.ShapeDtypeStruct((E_PAD, POS_DIM), jnp.float32)),
        mesh=plsc.VectorSubcoreMesh(core_axis_name="c", subcore_axis_name="s",
                                    num_cores=NC, num_subcores=NS),
        scratch_types=[
            pltpu.VMEM((ROWS_PER_W,), jnp.int32),
            pltpu.VMEM((ROWS_PER_W,), jnp.int32),
            pltpu.VMEM((NBUF, CHUNK, INPUT_DIM), jnp.float32),
            pltpu.VMEM((NBUF, CHUNK, POS_DIM), jnp.float32),
            pltpu.SemaphoreType.DMA,
            pltpu.SemaphoreType.DMA,
            pltpu.SemaphoreType.DMA,
            pltpu.SemaphoreType.DMA,
        ],
    )


def _mlp_body(a_ref, b_ref, w1a_ref, w1b_ref, b1_ref, w2_ref, b2_ref,
              w3_ref, b3_ref, o_ref):
    h = jnp.dot(a_ref[...], w1a_ref[...], preferred_element_type=jnp.float32)
    h = h + jnp.dot(b_ref[...], w1b_ref[...], preferred_element_type=jnp.float32)
    h = _gelu(h + b1_ref[...])
    h = _gelu(jnp.dot(h, w2_ref[...], preferred_element_type=jnp.float32)
              + b2_ref[...])
    h = jnp.dot(h, w3_ref[...], preferred_element_type=jnp.float32) + b3_ref[...]
    o_ref[...] = h


def _edge_mlp(h0a, h0b, W1a, W1b, b1, W2, b2, W3, b3):
    nblk = E_PAD // MLP_BLK
    full = lambda shape: pl.BlockSpec(shape, lambda i: (0, 0))
    return pl.pallas_call(
        _mlp_body,
        grid=(nblk,),
        in_specs=[
            pl.BlockSpec((MLP_BLK, INPUT_DIM), lambda i: (i, 0)),
            pl.BlockSpec((MLP_BLK, POS_DIM), lambda i: (i, 0)),
            full((INPUT_DIM, 512)), full((POS_DIM, 512)), full((1, 512)),
            full((512, 256)), full((1, 256)),
            full((256, 128)), full((1, 128)),
        ],
        out_specs=pl.BlockSpec((MLP_BLK, H3W), lambda i: (i, 0)),
        out_shape=jax.ShapeDtypeStruct((E_PAD, H3W), jnp.float32),
    )(h0a, h0b, W1a, W1b, b1, W2, b2, W3, b3)


CROWS = NROW // 128
CROWS_PER_TILE = 8


def _scatter_body(h3_hbm, qidx_hbm, zeros_hbm, sums_hbm, cnt_hbm,
                  qi_v, rows_v, cnt2_v, ridx_v, acc_sh, cntacc_sh,
                  fsem0, fsem1, asem0, asem1):
    fsem = [fsem0, fsem1]
    asem = [asem0, asem1]
    cid = lax.axis_index("c")
    sid = lax.axis_index("s")
    wid = sid * NC + cid
    wbase = wid * ROWS_PER_W
    my = pl.ds(sid * ROWS_PER_TILE, ROWS_PER_TILE)
    myc = pl.ds(sid * CROWS_PER_TILE, CROWS_PER_TILE)
    pltpu.sync_copy(zeros_hbm, acc_sh.at[my])
    pltpu.sync_copy(zeros_hbm.at[pl.ds(0, CROWS)], cnt2_v)

    @pl.when(sid < CROWS // CROWS_PER_TILE)
    def _():
        pltpu.sync_copy(zeros_hbm.at[pl.ds(0, CROWS_PER_TILE)],
                        cntacc_sh.at[myc])
    for j in range(CROWS // 16):
        ridx_v[pl.ds(j * 16, 16)] = lax.iota(jnp.int32, 16) + (j * 16)
    plsc.subcore_barrier()
    ones16 = jnp.ones((16,), jnp.float32)

    def f_copies(c, b):
        sl = pl.ds(wbase + c * CHUNK, CHUNK)
        return (pltpu.make_async_copy(qidx_hbm.at[sl], qi_v.at[b], fsem[b]),
                pltpu.make_async_copy(h3_hbm.at[sl], rows_v.at[b], fsem[b]))

    def a_copy(b):
        return pltpu.async_copy(rows_v.at[b], acc_sh.at[qi_v.at[b]], asem[b],
                                add=True)

    def a_waiter(b):
        pltpu.make_async_copy(rows_v.at[b], acc_sh.at[qi_v.at[b]],
                              asem[b]).wait()

    for b in range(NBUF):
        for cp in f_copies(b, b):
            cp.start()

    def step(i, _):
        for b in range(NBUF):
            c = i * NBUF + b
            for cp in f_copies(c, b):
                cp.wait()
            a_copy(b)
            for j in range(CHUNK // 16):
                idx16 = qi_v[b, pl.ds(j * 16, 16)]
                plsc.addupdate_scatter(
                    cnt2_v, [lax.shift_right_logical(idx16, 7),
                             lax.bitwise_and(idx16, 127)], ones16)

            @pl.when(c + NBUF < CHUNKS_PER_W)
            def _():
                a_waiter(b)
                for cp in f_copies(c + NBUF, b):
                    cp.start()

        return 0

    lax.fori_loop(0, CHUNKS_PER_W // NBUF, step, 0)
    for b in range(NBUF):
        a_waiter(b)
    pltpu.sync_copy(cnt2_v, cntacc_sh.at[ridx_v], add=True)
    plsc.subcore_barrier()
    pltpu.sync_copy(acc_sh.at[my], sums_hbm.at[cid].at[my])

    @pl.when(sid < CROWS // CROWS_PER_TILE)
    def _():
        pltpu.sync_copy(cntacc_sh.at[myc], cnt_hbm.at[cid].at[myc])


@functools.cache
def _scatter():
    return pl.kernel(
        _scatter_body,
        out_type=(jax.ShapeDtypeStruct((NC, NROW, H3W), jnp.float32),
                  jax.ShapeDtypeStruct((NC, CROWS, 128), jnp.float32)),
        mesh=plsc.VectorSubcoreMesh(core_axis_name="c", subcore_axis_name="s",
                                    num_cores=NC, num_subcores=NS),
        scratch_types=[
            pltpu.VMEM((NBUF, CHUNK), jnp.int32),
            pltpu.VMEM((NBUF, CHUNK, H3W), jnp.float32),
            pltpu.VMEM((CROWS, 128), jnp.float32),
            pltpu.VMEM((CROWS,), jnp.int32),
            pltpu.VMEM_SHARED((NROW, H3W), jnp.float32),
            pltpu.VMEM_SHARED((CROWS, 128), jnp.float32),
            pltpu.SemaphoreType.DMA,
            pltpu.SemaphoreType.DMA,
            pltpu.SemaphoreType.DMA,
            pltpu.SemaphoreType.DMA,
        ],
        compiler_params=pltpu.CompilerParams(needs_layout_passes=False),
    )


def _fin_body(p0_ref, p1_ref, c0_ref, c1_ref, P1_ref, pb1_ref, P2_ref,
              pb2_ref, o_ref):
    sums = p0_ref[...] + p1_ref[...]
    cnt = c0_ref[...] + c1_ref[...]
    mean = jnp.where(cnt > 0, sums / jnp.maximum(cnt, 1.0), 0.0)
    h = _gelu(jnp.dot(mean, P1_ref[...], preferred_element_type=jnp.float32)
              + pb1_ref[...])
    o_ref[...] = (jnp.dot(h, P2_ref[...], preferred_element_type=jnp.float32)
                  + pb2_ref[...])


def _finalize(p0, p1, c0, c1, P1, pb1, P2, pb2):
    nblk = N_QUERY // FIN_BLK
    full = lambda shape: pl.BlockSpec(shape, lambda i: (0, 0))
    return pl.pallas_call(
        _fin_body,
        grid=(nblk,),
        in_specs=[
            pl.BlockSpec((FIN_BLK, H3W), lambda i: (i, 0)),
            pl.BlockSpec((FIN_BLK, H3W), lambda i: (i, 0)),
            pl.BlockSpec((FIN_BLK, 1), lambda i: (i, 0)),
            pl.BlockSpec((FIN_BLK, 1), lambda i: (i, 0)),
            full((128, 256)), full((1, 256)), full((256, 4)), full((1, 4)),
        ],
        out_specs=pl.BlockSpec((FIN_BLK, 4), lambda i: (i, 0)),
        out_shape=jax.ShapeDtypeStruct((N_QUERY, 4), jnp.float32),
    )(p0, p1, c0, c1, P1, pb1, P2, pb2)


def kernel(x, query_pos, grid_to_query_edges, W1, b1, W2, b2, W3, b3,
           P1, pb1, P2, pb2):
    xf = x.reshape(-1, x.shape[-1])
    qp = query_pos / 100.0 - 1.0
    qp4 = jnp.concatenate([qp, jnp.ones((N_QUERY, 1), qp.dtype)], axis=1)
    qp_exp = jnp.repeat(qp4, 32, axis=1)

    pos = _pos_embed(qp_exp)

    pad = E_PAD - N_EDGES
    qidx = grid_to_query_edges[:, 0]
    gidx_p = jnp.concatenate([grid_to_query_edges[:, 1],
                              jnp.zeros((pad,), jnp.int32)])
    qidx_g = jnp.concatenate([qidx, jnp.zeros((pad,), jnp.int32)])
    qidx_s = jnp.concatenate([qidx, jnp.full((pad,), N_QUERY, jnp.int32)])

    h0a, h0b = _gather()(xf, pos, gidx_p, qidx_g)

    h3 = _edge_mlp(h0a, h0b, W1[:INPUT_DIM], W1[INPUT_DIM:],
                   b1.reshape(1, -1), W2, b2.reshape(1, -1),
                   W3, b3.reshape(1, -1))

    zeros = jnp.zeros((ROWS_PER_TILE, H3W), jnp.float32)
    sums_p, cnt_p = _scatter()(h3, qidx_s, zeros)

    return _finalize(sums_p[0], sums_p[1],
                     cnt_p[0].reshape(NROW, 1), cnt_p[1].reshape(NROW, 1),
                     P1, pb1.reshape(1, -1), P2, pb2.reshape(1, -1))


# speedup vs baseline: 3.6130x; 1.0113x over previous
import functools

import jax
import jax.numpy as jnp
from jax import lax
from jax.experimental import pallas as pl
from jax.experimental.pallas import tpu as pltpu
import jax.experimental.pallas.tpu_sc as plsc

INPUT_DIM = 128
HIDDEN = 32
POS_DIM = 4 * HIDDEN
N_QUERY = 10000
N_EDGES = 160000
SEQLEN = 32768

NC, NS = 2, 16
NW = NC * NS
CHUNK = 128
CHUNKS_PER_W = 5120 // CHUNK
E_PAD = NW * CHUNKS_PER_W * CHUNK
ROWS_PER_W = E_PAD // NW
H3W = 128
NROW = 10240
ROWS_PER_TILE = NROW // NS

MLP_BLK = 1024
FIN_BLK = 1000


def _gelu(x):
    return 0.5 * x * (1.0 + lax.erf(x * 0.7071067811865476))


def _pos_body(qp_ref, o_ref):
    k32 = lax.broadcasted_iota(jnp.int32, o_ref.shape, 1) % 32
    i = (k32 % 16).astype(jnp.float32)
    omega = jnp.exp(i * (-2.0 / 32.0 * jnp.log(10000.0)))
    ph = qp_ref[...] * omega
    o_ref[...] = jnp.where(k32 < 16, jnp.sin(ph), jnp.cos(ph))


def _pos_embed(qp_exp):
    return pl.pallas_call(
        _pos_body,
        out_shape=jax.ShapeDtypeStruct((N_QUERY, POS_DIM), jnp.float32),
    )(qp_exp)


NBUF = 2
PAIR_CHUNKS = 2 * CHUNKS_PER_W
NF = 58
PAIR_ROWS = PAIR_CHUNKS * CHUNK


def _gather_body(xf_hbm, pos_hbm, gidx_hbm, qidx_hbm, outa_hbm, outb_hbm,
                 gi_v, qi_v, ra_v, rb_v, gsem0, gsem1, wsem0, wsem1):
    gsem = [gsem0, gsem1]
    wsem = [wsem0, wsem1]
    cid = lax.axis_index("c")
    sid = lax.axis_index("s")
    wbase = sid * PAIR_ROWS + jnp.where(cid == 0, 0, NF * CHUNK)
    ntrips = jnp.where(cid == 0, NF, PAIR_CHUNKS - NF)

    @pl.when(cid == 0)
    def _():
        pltpu.sync_copy(gidx_hbm.at[pl.ds(wbase, NF * CHUNK)],
                        gi_v.at[pl.ds(0, NF * CHUNK)])
        pltpu.sync_copy(qidx_hbm.at[pl.ds(wbase, NF * CHUNK)],
                        qi_v.at[pl.ds(0, NF * CHUNK)])

    @pl.when(cid == 1)
    def _():
        pltpu.sync_copy(gidx_hbm.at[pl.ds(wbase, (PAIR_CHUNKS - NF) * CHUNK)],
                        gi_v.at[pl.ds(0, (PAIR_CHUNKS - NF) * CHUNK)])
        pltpu.sync_copy(qidx_hbm.at[pl.ds(wbase, (PAIR_CHUNKS - NF) * CHUNK)],
                        qi_v.at[pl.ds(0, (PAIR_CHUNKS - NF) * CHUNK)])

    def g_copies(c, b):
        sl = pl.ds(c * CHUNK, CHUNK)
        return (pltpu.make_async_copy(xf_hbm.at[gi_v.at[sl]], ra_v.at[b],
                                      gsem[b]),
                pltpu.make_async_copy(pos_hbm.at[qi_v.at[sl]], rb_v.at[b],
                                      gsem[b]))

    def w_copies(c, b):
        sl = pl.ds(wbase + c * CHUNK, CHUNK)
        return (pltpu.make_async_copy(ra_v.at[b], outa_hbm.at[sl], wsem[b]),
                pltpu.make_async_copy(rb_v.at[b], outb_hbm.at[sl], wsem[b]))

    def start(c, b):
        for cp in g_copies(c, b):
            cp.start()

    def drain_gather(c, b):
        for cp in g_copies(c, b):
            cp.wait()

    def writeback(c, b):
        for cp in w_copies(c, b):
            cp.start()

    def drain_write(c, b):
        for cp in w_copies(c, b):
            cp.wait()

    for b in range(NBUF):
        start(b, b)

    def step(i, _):
        for b in range(NBUF):
            c = i * NBUF + b
            drain_gather(c, b)
            writeback(c, b)

            @pl.when(c + NBUF < ntrips)
            def _():
                drain_write(c, b)
                start(c + NBUF, b)

        return 0

    lax.fori_loop(0, ntrips // NBUF, step, 0)
    for b in range(NBUF):
        drain_write(ntrips - NBUF + b, b)


@functools.cache
def _gather():
    return pl.kernel(
        _gather_body,
        out_type=(jax.ShapeDtypeStruct((E_PAD, INPUT_DIM), jnp.float32),
                  jax.ShapeDtypeStruct((E_PAD, POS_DIM), jnp.float32)),
        mesh=plsc.VectorSubcoreMesh(core_axis_name="c", subcore_axis_name="s",
                                    num_cores=NC, num_subcores=NS),
        scratch_types=[
            pltpu.VMEM((NF * CHUNK,), jnp.int32),
            pltpu.VMEM((NF * CHUNK,), jnp.int32),
            pltpu.VMEM((NBUF, CHUNK, INPUT_DIM), jnp.float32),
            pltpu.VMEM((NBUF, CHUNK, POS_DIM), jnp.float32),
            pltpu.SemaphoreType.DMA,
            pltpu.SemaphoreType.DMA,
            pltpu.SemaphoreType.DMA,
            pltpu.SemaphoreType.DMA,
        ],
    )


def _mlp_body(a_ref, b_ref, w1a_ref, w1b_ref, b1_ref, w2_ref, b2_ref,
              w3_ref, b3_ref, o_ref):
    h = jnp.dot(a_ref[...], w1a_ref[...], preferred_element_type=jnp.float32)
    h = h + jnp.dot(b_ref[...], w1b_ref[...], preferred_element_type=jnp.float32)
    h = _gelu(h + b1_ref[...])
    h = _gelu(jnp.dot(h, w2_ref[...], preferred_element_type=jnp.float32)
              + b2_ref[...])
    h = jnp.dot(h, w3_ref[...], preferred_element_type=jnp.float32) + b3_ref[...]
    o_ref[...] = h


def _edge_mlp(h0a, h0b, W1a, W1b, b1, W2, b2, W3, b3):
    nblk = E_PAD // MLP_BLK
    full = lambda shape: pl.BlockSpec(shape, lambda i: (0, 0))
    return pl.pallas_call(
        _mlp_body,
        grid=(nblk,),
        in_specs=[
            pl.BlockSpec((MLP_BLK, INPUT_DIM), lambda i: (i, 0)),
            pl.BlockSpec((MLP_BLK, POS_DIM), lambda i: (i, 0)),
            full((INPUT_DIM, 512)), full((POS_DIM, 512)), full((1, 512)),
            full((512, 256)), full((1, 256)),
            full((256, 128)), full((1, 128)),
        ],
        out_specs=pl.BlockSpec((MLP_BLK, H3W), lambda i: (i, 0)),
        out_shape=jax.ShapeDtypeStruct((E_PAD, H3W), jnp.float32),
    )(h0a, h0b, W1a, W1b, b1, W2, b2, W3, b3)


CROWS = NROW // 128
CROWS_PER_TILE = 8


def _scatter_body(h3_hbm, qidx_hbm, zeros_hbm, sums_hbm, cnt_hbm,
                  qi_v, rows_v, cnt2_v, ridx_v, acc_sh, cntacc_sh,
                  fsem0, fsem1, asem0, asem1):
    fsem = [fsem0, fsem1]
    asem = [asem0, asem1]
    cid = lax.axis_index("c")
    sid = lax.axis_index("s")
    wid = sid * NC + cid
    wbase = wid * ROWS_PER_W
    my = pl.ds(sid * ROWS_PER_TILE, ROWS_PER_TILE)
    myc = pl.ds(sid * CROWS_PER_TILE, CROWS_PER_TILE)
    pltpu.sync_copy(zeros_hbm, acc_sh.at[my])
    pltpu.sync_copy(zeros_hbm.at[pl.ds(0, CROWS)], cnt2_v)

    @pl.when(sid < CROWS // CROWS_PER_TILE)
    def _():
        pltpu.sync_copy(zeros_hbm.at[pl.ds(0, CROWS_PER_TILE)],
                        cntacc_sh.at[myc])
    for j in range(CROWS // 16):
        ridx_v[pl.ds(j * 16, 16)] = lax.iota(jnp.int32, 16) + (j * 16)
    plsc.subcore_barrier()
    ones16 = jnp.ones((16,), jnp.float32)

    def f_copies(c, b):
        sl = pl.ds(wbase + c * CHUNK, CHUNK)
        return (pltpu.make_async_copy(qidx_hbm.at[sl], qi_v.at[b], fsem[b]),
                pltpu.make_async_copy(h3_hbm.at[sl], rows_v.at[b], fsem[b]))

    def a_copy(b):
        return pltpu.async_copy(rows_v.at[b], acc_sh.at[qi_v.at[b]], asem[b],
                                add=True)

    def a_waiter(b):
        pltpu.make_async_copy(rows_v.at[b], acc_sh.at[qi_v.at[b]],
                              asem[b]).wait()

    for b in range(NBUF):
        for cp in f_copies(b, b):
            cp.start()

    def step(i, _):
        for b in range(NBUF):
            c = i * NBUF + b
            for cp in f_copies(c, b):
                cp.wait()
            a_copy(b)
            for j in range(CHUNK // 16):
                idx16 = qi_v[b, pl.ds(j * 16, 16)]
                plsc.addupdate_scatter(
                    cnt2_v, [lax.shift_right_logical(idx16, 7),
                             lax.bitwise_and(idx16, 127)], ones16)

            @pl.when(c + NBUF < CHUNKS_PER_W)
            def _():
                a_waiter(b)
                for cp in f_copies(c + NBUF, b):
                    cp.start()

        return 0

    lax.fori_loop(0, CHUNKS_PER_W // NBUF, step, 0)
    for b in range(NBUF):
        a_waiter(b)
    pltpu.sync_copy(cnt2_v, cntacc_sh.at[ridx_v], add=True)
    plsc.subcore_barrier()
    pltpu.sync_copy(acc_sh.at[my], sums_hbm.at[cid].at[my])

    @pl.when(sid < CROWS // CROWS_PER_TILE)
    def _():
        pltpu.sync_copy(cntacc_sh.at[myc], cnt_hbm.at[cid].at[myc])


@functools.cache
def _scatter():
    return pl.kernel(
        _scatter_body,
        out_type=(jax.ShapeDtypeStruct((NC, NROW, H3W), jnp.float32),
                  jax.ShapeDtypeStruct((NC, CROWS, 128), jnp.float32)),
        mesh=plsc.VectorSubcoreMesh(core_axis_name="c", subcore_axis_name="s",
                                    num_cores=NC, num_subcores=NS),
        scratch_types=[
            pltpu.VMEM((NBUF, CHUNK), jnp.int32),
            pltpu.VMEM((NBUF, CHUNK, H3W), jnp.float32),
            pltpu.VMEM((CROWS, 128), jnp.float32),
            pltpu.VMEM((CROWS,), jnp.int32),
            pltpu.VMEM_SHARED((NROW, H3W), jnp.float32),
            pltpu.VMEM_SHARED((CROWS, 128), jnp.float32),
            pltpu.SemaphoreType.DMA,
            pltpu.SemaphoreType.DMA,
            pltpu.SemaphoreType.DMA,
            pltpu.SemaphoreType.DMA,
        ],
        compiler_params=pltpu.CompilerParams(needs_layout_passes=False),
    )


def _fin_body(p0_ref, p1_ref, c0_ref, c1_ref, P1_ref, pb1_ref, P2_ref,
              pb2_ref, o_ref):
    sums = p0_ref[...] + p1_ref[...]
    cnt = c0_ref[...] + c1_ref[...]
    mean = jnp.where(cnt > 0, sums / jnp.maximum(cnt, 1.0), 0.0)
    h = _gelu(jnp.dot(mean, P1_ref[...], preferred_element_type=jnp.float32)
              + pb1_ref[...])
    o_ref[...] = (jnp.dot(h, P2_ref[...], preferred_element_type=jnp.float32)
                  + pb2_ref[...])


def _finalize(p0, p1, c0, c1, P1, pb1, P2, pb2):
    nblk = N_QUERY // FIN_BLK
    full = lambda shape: pl.BlockSpec(shape, lambda i: (0, 0))
    return pl.pallas_call(
        _fin_body,
        grid=(nblk,),
        in_specs=[
            pl.BlockSpec((FIN_BLK, H3W), lambda i: (i, 0)),
            pl.BlockSpec((FIN_BLK, H3W), lambda i: (i, 0)),
            pl.BlockSpec((FIN_BLK, 1), lambda i: (i, 0)),
            pl.BlockSpec((FIN_BLK, 1), lambda i: (i, 0)),
            full((128, 256)), full((1, 256)), full((256, 4)), full((1, 4)),
        ],
        out_specs=pl.BlockSpec((FIN_BLK, 4), lambda i: (i, 0)),
        out_shape=jax.ShapeDtypeStruct((N_QUERY, 4), jnp.float32),
    )(p0, p1, c0, c1, P1, pb1, P2, pb2)


def kernel(x, query_pos, grid_to_query_edges, W1, b1, W2, b2, W3, b3,
           P1, pb1, P2, pb2):
    xf = x.reshape(-1, x.shape[-1])
    qp = query_pos / 100.0 - 1.0
    qp4 = jnp.concatenate([qp, jnp.ones((N_QUERY, 1), qp.dtype)], axis=1)
    qp_exp = jnp.repeat(qp4, 32, axis=1)

    pos = _pos_embed(qp_exp)

    pad = E_PAD - N_EDGES
    qidx = grid_to_query_edges[:, 0]
    gidx_p = jnp.concatenate([grid_to_query_edges[:, 1],
                              jnp.zeros((pad,), jnp.int32)])
    qidx_g = jnp.concatenate([qidx, jnp.zeros((pad,), jnp.int32)])
    qidx_s = jnp.concatenate([qidx, jnp.full((pad,), N_QUERY, jnp.int32)])

    h0a, h0b = _gather()(xf, pos, gidx_p, qidx_g)

    h3 = _edge_mlp(h0a, h0b, W1[:INPUT_DIM], W1[INPUT_DIM:],
                   b1.reshape(1, -1), W2, b2.reshape(1, -1),
                   W3, b3.reshape(1, -1))

    zeros = jnp.zeros((ROWS_PER_TILE, H3W), jnp.float32)
    sums_p, cnt_p = _scatter()(h3, qidx_s, zeros)

    return _finalize(sums_p[0], sums_p[1],
                     cnt_p[0].reshape(NROW, 1), cnt_p[1].reshape(NROW, 1),
                     P1, pb1.reshape(1, -1), P2, pb2.reshape(1, -1))
